# baseline (speedup 1.0000x reference)
import functools

import jax
import jax.numpy as jnp
from jax import lax
from jax.experimental import pallas as pl
from jax.experimental.pallas import tpu as pltpu
from jax.experimental.pallas import tpu_sc as plsc


def _table_body(embed_ref, ln_w_ref, ln_b_ref, w_ref, b_ref, table_ref):
    h = embed_ref[...]
    mean = jnp.mean(h, axis=1, keepdims=True)
    var = jnp.mean(jnp.square(h - mean), axis=1, keepdims=True)
    hn = (h - mean) / jnp.sqrt(var + 1e-5) * ln_w_ref[...] + ln_b_ref[...]
    table_ref[...] = (
        lax.dot_general(hn, w_ref[...], (((1,), (1,)), ((), ())),
                        preferred_element_type=jnp.float32)
        + b_ref[...]
    )


def _make_table(embed, ln_w, ln_b, W, b):
    V, E = embed.shape
    return pl.pallas_call(
        _table_body,
        out_shape=jax.ShapeDtypeStruct((V, V), jnp.float32),
    )(embed, ln_w.reshape(1, E), ln_b.reshape(1, E), W, b.reshape(1, V))


def _gather_body(n_pairs, chunk, table_hbm, idx_hbm, out_hbm,
                 idx_v, rows0, rows1, gsem0, gsem1, wsem0, wsem1):
    info = plsc.get_sparse_core_info()
    nc = info.num_cores
    sid = lax.axis_index("s")
    wid = sid * nc + lax.axis_index("c")
    per_tile = 2 * n_pairs * chunk
    base = wid * per_tile

    pltpu.sync_copy(idx_hbm.at[pl.ds(base, per_tile)], idx_v)

    pltpu.async_copy(table_hbm.at[idx_v.at[pl.ds(0, chunk)]], rows0, gsem0)
    pltpu.async_copy(table_hbm.at[idx_v.at[pl.ds(chunk, chunk)]], rows1, gsem1)

    def step(i, _):
        off0 = (2 * i) * chunk
        off1 = off0 + chunk

        pltpu.make_async_copy(
            table_hbm.at[idx_v.at[pl.ds(off0, chunk)]], rows0, gsem0).wait()
        pltpu.async_copy(rows0, out_hbm.at[pl.ds(base + off0, chunk)], wsem0)
        pltpu.make_async_copy(
            table_hbm.at[idx_v.at[pl.ds(off1, chunk)]], rows1, gsem1).wait()
        pltpu.async_copy(rows1, out_hbm.at[pl.ds(base + off1, chunk)], wsem1)

        @pl.when(i + 1 < n_pairs)
        def _prefetch():
            nxt0 = off0 + 2 * chunk
            nxt1 = off1 + 2 * chunk
            pltpu.make_async_copy(
                rows0, out_hbm.at[pl.ds(base + off0, chunk)], wsem0).wait()
            pltpu.async_copy(table_hbm.at[idx_v.at[pl.ds(nxt0, chunk)]],
                             rows0, gsem0)
            pltpu.make_async_copy(
                rows1, out_hbm.at[pl.ds(base + off1, chunk)], wsem1).wait()
            pltpu.async_copy(table_hbm.at[idx_v.at[pl.ds(nxt1, chunk)]],
                             rows1, gsem1)
        return 0

    lax.fori_loop(0, n_pairs, step, 0)
    pltpu.make_async_copy(rows0, out_hbm.at[pl.ds(base, chunk)], wsem0).wait()
    pltpu.make_async_copy(rows1, out_hbm.at[pl.ds(base, chunk)], wsem1).wait()


def _gather_rows(table, ids):
    V, D = table.shape
    (B,) = ids.shape
    info = plsc.get_sparse_core_info()
    nw = info.num_cores * info.num_subcores
    per_tile = B // nw
    chunk = 40
    n_pairs = per_tile // (2 * chunk)
    mesh = plsc.VectorSubcoreMesh(core_axis_name="c", subcore_axis_name="s")
    grab = functools.partial(
        pl.kernel,
        mesh=mesh,
        out_type=jax.ShapeDtypeStruct((B, D), jnp.float32),
        scratch_types=[
            pltpu.VMEM((per_tile,), jnp.int32),
            pltpu.VMEM((chunk, D), jnp.float32),
            pltpu.VMEM((chunk, D), jnp.float32),
            pltpu.SemaphoreType.DMA,
            pltpu.SemaphoreType.DMA,
            pltpu.SemaphoreType.DMA,
            pltpu.SemaphoreType.DMA,
        ],
        compiler_params=pltpu.CompilerParams(use_tc_tiling_on_sc=False),
    )(functools.partial(_gather_body, n_pairs, chunk))
    return grab(table, ids)


def kernel(input_ids, embed, ln_w, ln_b, W, b):
    Bt, Lt = input_ids.shape
    V, _ = embed.shape
    table = _make_table(embed, ln_w, ln_b, W, b)
    ids = input_ids.reshape(-1).astype(jnp.int32)
    out = _gather_rows(table, ids)
    return out.reshape(Bt, Lt, V)


# speedup vs baseline: 1.0027x; 1.0027x over previous
import functools

import jax
import jax.numpy as jnp
from jax import lax
from jax.experimental import pallas as pl
from jax.experimental.pallas import tpu as pltpu
from jax.experimental.pallas import tpu_sc as plsc


def _table_body(embed_ref, ln_w_ref, ln_b_ref, w_ref, b_ref, table_ref):
    h = embed_ref[...]
    mean = jnp.mean(h, axis=1, keepdims=True)
    var = jnp.mean(jnp.square(h - mean), axis=1, keepdims=True)
    hn = (h - mean) / jnp.sqrt(var + 1e-5) * ln_w_ref[...] + ln_b_ref[...]
    table_ref[...] = (
        lax.dot_general(hn, w_ref[...], (((1,), (1,)), ((), ())),
                        preferred_element_type=jnp.float32)
        + b_ref[...]
    )


def _make_table(embed, ln_w, ln_b, W, b):
    V, E = embed.shape
    return pl.pallas_call(
        _table_body,
        out_shape=jax.ShapeDtypeStruct((V, V), jnp.float32),
    )(embed, ln_w.reshape(1, E), ln_b.reshape(1, E), W, b.reshape(1, V))


_NBUF = 4


def _gather_body(n_groups, chunk, table_hbm, idx_hbm, out_hbm, idx_v, *scr):
    bufs = scr[:_NBUF]
    gsems = scr[_NBUF:2 * _NBUF]
    wsems = scr[2 * _NBUF:3 * _NBUF]
    info = plsc.get_sparse_core_info()
    nc = info.num_cores
    sid = lax.axis_index("s")
    wid = sid * nc + lax.axis_index("c")
    per_tile = n_groups * _NBUF * chunk
    base = wid * per_tile

    pltpu.sync_copy(idx_hbm.at[pl.ds(base, per_tile)], idx_v)

    for b in range(_NBUF):
        pltpu.async_copy(
            table_hbm.at[idx_v.at[pl.ds(b * chunk, chunk)]], bufs[b], gsems[b])

    def step(g, _):
        goff = g * _NBUF * chunk

        for b in range(_NBUF):
            off = goff + b * chunk
            pltpu.make_async_copy(
                table_hbm.at[idx_v.at[pl.ds(off, chunk)]],
                bufs[b], gsems[b]).wait()
            pltpu.async_copy(
                bufs[b], out_hbm.at[pl.ds(base + off, chunk)], wsems[b])

        @pl.when(g + 1 < n_groups)
        def _prefetch():
            for b in range(_NBUF):
                off = goff + b * chunk
                pltpu.make_async_copy(
                    bufs[b], out_hbm.at[pl.ds(base + off, chunk)],
                    wsems[b]).wait()
                pltpu.async_copy(
                    table_hbm.at[idx_v.at[pl.ds(off + _NBUF * chunk, chunk)]],
                    bufs[b], gsems[b])
        return 0

    lax.fori_loop(0, n_groups, step, 0)
    for b in range(_NBUF):
        pltpu.make_async_copy(
            bufs[b], out_hbm.at[pl.ds(base, chunk)], wsems[b]).wait()


def _gather_rows(table, ids):
    V, D = table.shape
    (B,) = ids.shape
    info = plsc.get_sparse_core_info()
    nw = info.num_cores * info.num_subcores
    per_tile = B // nw
    chunk = 16
    n_groups = per_tile // (_NBUF * chunk)
    mesh = plsc.VectorSubcoreMesh(core_axis_name="c", subcore_axis_name="s")
    grab = functools.partial(
        pl.kernel,
        mesh=mesh,
        out_type=jax.ShapeDtypeStruct((B, D), jnp.float32),
        scratch_types=(
            [pltpu.VMEM((per_tile,), jnp.int32)]
            + [pltpu.VMEM((chunk, D), jnp.float32)] * _NBUF
            + [pltpu.SemaphoreType.DMA] * (2 * _NBUF)
        ),
        compiler_params=pltpu.CompilerParams(use_tc_tiling_on_sc=False),
    )(functools.partial(_gather_body, n_groups, chunk))
    return grab(table, ids)


def kernel(input_ids, embed, ln_w, ln_b, W, b):
    Bt, Lt = input_ids.shape
    V, _ = embed.shape
    table = _make_table(embed, ln_w, ln_b, W, b)
    ids = input_ids.reshape(-1).astype(jnp.int32)
    out = _gather_rows(table, ids)
    return out.reshape(Bt, Lt, V)
